# baseline (speedup 1.0000x reference)
import functools

import jax
import jax.numpy as jnp
from jax import lax
from jax.experimental import pallas as pl
from jax.experimental.pallas import tpu as pltpu
from jax.experimental.pallas import tpu_sc as plsc

B, L, D, V = 64, 512, 768, 100000
N = B * L
NC, NS = 2, 16
NW = NC * NS
TOK_W = N // NW
CHUNK = 64
NCHUNK = TOK_W // CHUNK

BLK = 1024
NBLK = N // BLK
K_CAT = 512 + 3 + 21


def _sc_gather_word(src_w, word_table):
    mesh = plsc.VectorSubcoreMesh(core_axis_name="c", subcore_axis_name="s")

    @functools.partial(
        pl.kernel,
        out_type=jax.ShapeDtypeStruct((N, D), jnp.float32),
        mesh=mesh,
        scratch_types=[
            pltpu.VMEM((NCHUNK, CHUNK), jnp.int32),
            pltpu.VMEM((CHUNK, D), jnp.float32),
            pltpu.VMEM((CHUNK, D), jnp.float32),
            pltpu.SemaphoreType.DMA,
            pltpu.SemaphoreType.DMA,
        ],
    )
    def gather_kernel(src_hbm, tab_hbm, out_hbm, idx_v, buf0, buf1, sem0, sem1):
        wid = lax.axis_index("s") * NC + lax.axis_index("c")
        base = wid * TOK_W
        pltpu.sync_copy(src_hbm.at[wid], idx_v)
        bufs = (buf0, buf1)
        sems = (sem0, sem1)

        pltpu.async_copy(tab_hbm.at[idx_v.at[0]], buf0, sem0)

        def body(j, _):
            slot = lax.rem(j, 2)
            nslot = lax.rem(j + 1, 2)

            @pl.when(j + 1 < NCHUNK)
            def _():
                def fire(s):
                    pltpu.async_copy(tab_hbm.at[idx_v.at[j + 1]], bufs[s], sems[s])
                lax.cond(nslot == 0, lambda: fire(0), lambda: fire(1))

            def drain(s):
                pltpu.make_async_copy(tab_hbm.at[idx_v.at[j]], bufs[s], sems[s]).wait()
                pltpu.sync_copy(bufs[s], out_hbm.at[pl.ds(base + j * CHUNK, CHUNK)])
            lax.cond(slot == 0, lambda: drain(0), lambda: drain(1))
            return 0

        lax.fori_loop(0, NCHUNK, body, 0)

    return gather_kernel(src_w, word_table)


def _tc_body(g_r, pos_r, seg_r, typ_r, tab_r, gam_r, bet_r, out_r):
    posb = pos_r[0]
    segb = seg_r[0]
    typb = typ_r[0]
    k_iota = lax.broadcasted_iota(jnp.int32, (K_CAT, BLK), 0)
    oh = ((k_iota == posb) | (k_iota == segb + 512) | (k_iota == typb + 515))
    oh = oh.astype(jnp.bfloat16)
    small = lax.dot_general(oh, tab_r[...], (((0,), (0,)), ((), ())),
                            preferred_element_type=jnp.float32)
    x = g_r[...] + small
    mean = jnp.mean(x, axis=1, keepdims=True)
    xc = x - mean
    var = jnp.mean(xc * xc, axis=1, keepdims=True)
    y = xc * lax.rsqrt(var + 1e-6)
    out_r[...] = y * gam_r[...] + bet_r[...]


def _tc_fused(g, pos_i, seg_i, typ_i, cat_tab, gamma, beta):
    return pl.pallas_call(
        _tc_body,
        grid=(NBLK,),
        in_specs=[
            pl.BlockSpec((BLK, D), lambda i: (i, 0)),
            pl.BlockSpec((1, 1, BLK), lambda i: (i, 0, 0)),
            pl.BlockSpec((1, 1, BLK), lambda i: (i, 0, 0)),
            pl.BlockSpec((1, 1, BLK), lambda i: (i, 0, 0)),
            pl.BlockSpec((K_CAT, D), lambda i: (0, 0)),
            pl.BlockSpec((1, D), lambda i: (0, 0)),
            pl.BlockSpec((1, D), lambda i: (0, 0)),
        ],
        out_specs=pl.BlockSpec((BLK, D), lambda i: (i, 0)),
        out_shape=jax.ShapeDtypeStruct((N, D), jnp.float32),
    )(g, pos_i, seg_i, typ_i, cat_tab, gamma, beta)


def kernel(ids, src, seg, type, concept_ent_pairs, edge_idx, pos, need_gnn,
           word_table, token_type_table, pos_table, seg_table, gamma, beta):
    src_w = src.reshape(NW, NCHUNK, CHUNK).astype(jnp.int32)
    g = _sc_gather_word(src_w, word_table)

    cat_tab = jnp.concatenate([pos_table, seg_table, token_type_table],
                              axis=0).astype(jnp.bfloat16)
    pos_i = pos.reshape(NBLK, 1, BLK).astype(jnp.int32)
    seg_i = seg.reshape(NBLK, 1, BLK).astype(jnp.int32)
    typ_i = type.reshape(NBLK, 1, BLK).astype(jnp.int32)
    out = _tc_fused(g, pos_i, seg_i, typ_i, cat_tab,
                    gamma.reshape(1, D), beta.reshape(1, D))
    return out.reshape(B, L, D)


# speedup vs baseline: 1.0410x; 1.0410x over previous
import functools

import jax
import jax.numpy as jnp
from jax import lax
from jax.experimental import pallas as pl
from jax.experimental.pallas import tpu as pltpu
from jax.experimental.pallas import tpu_sc as plsc

B, L, D, V = 64, 512, 768, 100000
N = B * L
NC, NS = 2, 16
NW = NC * NS
TOK_W = N // NW
CHUNK = 64
NCHUNK = TOK_W // CHUNK

BLK = 2048
NBLK = N // BLK
K_CAT = 512 + 3 + 21


def _sc_gather_word(src_w, word_table):
    mesh = plsc.VectorSubcoreMesh(core_axis_name="c", subcore_axis_name="s")

    @functools.partial(
        pl.kernel,
        out_type=jax.ShapeDtypeStruct((N, D), jnp.float32),
        mesh=mesh,
        scratch_types=[
            pltpu.VMEM((NCHUNK, CHUNK), jnp.int32),
            pltpu.VMEM((CHUNK, D), jnp.float32),
            pltpu.VMEM((CHUNK, D), jnp.float32),
            pltpu.SemaphoreType.DMA,
            pltpu.SemaphoreType.DMA,
        ],
    )
    def gather_kernel(src_hbm, tab_hbm, out_hbm, idx_v, buf0, buf1, sem0, sem1):
        wid = lax.axis_index("s") * NC + lax.axis_index("c")
        base = wid * TOK_W
        pltpu.sync_copy(src_hbm.at[wid], idx_v)
        bufs = (buf0, buf1)
        sems = (sem0, sem1)

        pltpu.async_copy(tab_hbm.at[idx_v.at[0]], buf0, sem0)

        def body(j, _):
            slot = lax.rem(j, 2)
            nslot = lax.rem(j + 1, 2)

            @pl.when(j + 1 < NCHUNK)
            def _():
                def fire(s):
                    pltpu.async_copy(tab_hbm.at[idx_v.at[j + 1]], bufs[s], sems[s])
                lax.cond(nslot == 0, lambda: fire(0), lambda: fire(1))

            def drain(s):
                pltpu.make_async_copy(tab_hbm.at[idx_v.at[j]], bufs[s], sems[s]).wait()
                pltpu.sync_copy(bufs[s], out_hbm.at[pl.ds(base + j * CHUNK, CHUNK)])
            lax.cond(slot == 0, lambda: drain(0), lambda: drain(1))
            return 0

        lax.fori_loop(0, NCHUNK, body, 0)

    return gather_kernel(src_w, word_table)


def _tc_body(g_r, pos_r, seg_r, typ_r, tab_r, gam_r, bet_r, out_r):
    posb = pos_r[0]
    segb = seg_r[0]
    typb = typ_r[0]
    k_iota = lax.broadcasted_iota(jnp.int32, (K_CAT, BLK), 0)
    oh = ((k_iota == posb) | (k_iota == segb + 512) | (k_iota == typb + 515))
    oh = oh.astype(jnp.bfloat16)
    small = lax.dot_general(oh, tab_r[...], (((0,), (0,)), ((), ())),
                            preferred_element_type=jnp.float32)
    x = g_r[...] + small
    mean = jnp.mean(x, axis=1, keepdims=True)
    xc = x - mean
    var = jnp.mean(xc * xc, axis=1, keepdims=True)
    y = xc * lax.rsqrt(var + 1e-6)
    out_r[...] = y * gam_r[...] + bet_r[...]


def _tc_fused(g, pos_i, seg_i, typ_i, cat_tab, gamma, beta):
    return pl.pallas_call(
        _tc_body,
        grid=(NBLK,),
        in_specs=[
            pl.BlockSpec((BLK, D), lambda i: (i, 0)),
            pl.BlockSpec((1, 1, BLK), lambda i: (i, 0, 0)),
            pl.BlockSpec((1, 1, BLK), lambda i: (i, 0, 0)),
            pl.BlockSpec((1, 1, BLK), lambda i: (i, 0, 0)),
            pl.BlockSpec((K_CAT, D), lambda i: (0, 0)),
            pl.BlockSpec((1, D), lambda i: (0, 0)),
            pl.BlockSpec((1, D), lambda i: (0, 0)),
        ],
        out_specs=pl.BlockSpec((BLK, D), lambda i: (i, 0)),
        out_shape=jax.ShapeDtypeStruct((N, D), jnp.float32),
    )(g, pos_i, seg_i, typ_i, cat_tab, gamma, beta)


def kernel(ids, src, seg, type, concept_ent_pairs, edge_idx, pos, need_gnn,
           word_table, token_type_table, pos_table, seg_table, gamma, beta):
    src_w = src.reshape(NW, NCHUNK, CHUNK).astype(jnp.int32)
    g = _sc_gather_word(src_w, word_table)

    cat_tab = jnp.concatenate([pos_table, seg_table, token_type_table],
                              axis=0).astype(jnp.bfloat16)
    pos_i = pos.reshape(NBLK, 1, BLK).astype(jnp.int32)
    seg_i = seg.reshape(NBLK, 1, BLK).astype(jnp.int32)
    typ_i = type.reshape(NBLK, 1, BLK).astype(jnp.int32)
    out = _tc_fused(g, pos_i, seg_i, typ_i, cat_tab,
                    gamma.reshape(1, D), beta.reshape(1, D))
    return out.reshape(B, L, D)


# speedup vs baseline: 1.0693x; 1.0272x over previous
import functools

import jax
import jax.numpy as jnp
from jax import lax
from jax.experimental import pallas as pl
from jax.experimental.pallas import tpu as pltpu
from jax.experimental.pallas import tpu_sc as plsc

B, L, D, V = 64, 512, 768, 100000
N = B * L
NC, NS = 2, 16
NW = NC * NS
CHUNK = 64

S = 4
NSL = N // S
TOK_W = NSL // NW
NCHUNK = TOK_W // CHUNK

BLK = 2048
BLKS_S = NSL // BLK
K_CAT = 512 + 3 + 21


def _sc_gather_word(src_w, word_table, out_rows):
    mesh = plsc.VectorSubcoreMesh(core_axis_name="c", subcore_axis_name="s")

    @functools.partial(
        pl.kernel,
        out_type=jax.ShapeDtypeStruct((out_rows, D), jnp.float32),
        mesh=mesh,
        scratch_types=[
            pltpu.VMEM((NCHUNK, CHUNK), jnp.int32),
            pltpu.VMEM((CHUNK, D), jnp.float32),
            pltpu.VMEM((CHUNK, D), jnp.float32),
            pltpu.SemaphoreType.DMA,
            pltpu.SemaphoreType.DMA,
        ],
    )
    def gather_kernel(src_hbm, tab_hbm, out_hbm, idx_v, buf0, buf1, sem0, sem1):
        wid = lax.axis_index("s") * NC + lax.axis_index("c")
        base = wid * TOK_W
        pltpu.sync_copy(src_hbm.at[wid], idx_v)
        bufs = (buf0, buf1)
        sems = (sem0, sem1)

        pltpu.async_copy(tab_hbm.at[idx_v.at[0]], buf0, sem0)

        def body(j, _):
            slot = lax.rem(j, 2)
            nslot = lax.rem(j + 1, 2)

            @pl.when(j + 1 < NCHUNK)
            def _():
                def fire(s):
                    pltpu.async_copy(tab_hbm.at[idx_v.at[j + 1]], bufs[s], sems[s])
                lax.cond(nslot == 0, lambda: fire(0), lambda: fire(1))

            def drain(s):
                pltpu.make_async_copy(tab_hbm.at[idx_v.at[j]], bufs[s], sems[s]).wait()
                pltpu.sync_copy(bufs[s], out_hbm.at[pl.ds(base + j * CHUNK, CHUNK)])
            lax.cond(slot == 0, lambda: drain(0), lambda: drain(1))
            return 0

        lax.fori_loop(0, NCHUNK, body, 0)

    return gather_kernel(src_w, word_table)


def _emb_ln(g, posb, segb, typb, tab, gam, bet):
    k_iota = lax.broadcasted_iota(jnp.int32, (K_CAT, BLK), 0)
    oh = ((k_iota == posb) | (k_iota == segb + 512) | (k_iota == typb + 515))
    oh = oh.astype(jnp.bfloat16)
    small = lax.dot_general(oh, tab, (((0,), (0,)), ((), ())),
                            preferred_element_type=jnp.float32)
    x = g + small
    mean = jnp.mean(x, axis=1, keepdims=True)
    xc = x - mean
    var = jnp.mean(xc * xc, axis=1, keepdims=True)
    y = xc * lax.rsqrt(var + 1e-6)
    return y * gam + bet


def _tc_body0(g_r, pos_r, seg_r, typ_r, tab_r, gam_r, bet_r, out_r):
    out_r[...] = _emb_ln(g_r[...], pos_r[0], seg_r[0], typ_r[0],
                         tab_r[...], gam_r[...], bet_r[...])


def _tc_body_rest(buf_r, g_r, pos_r, seg_r, typ_r, tab_r, gam_r, bet_r, out_r):
    del buf_r
    out_r[...] = _emb_ln(g_r[...], pos_r[0], seg_r[0], typ_r[0],
                         tab_r[...], gam_r[...], bet_r[...])


_ID_SPEC = pl.BlockSpec((1, 1, BLK), lambda i: (i, 0, 0))
_TAB_SPECS = [
    pl.BlockSpec((K_CAT, D), lambda i: (0, 0)),
    pl.BlockSpec((1, D), lambda i: (0, 0)),
    pl.BlockSpec((1, D), lambda i: (0, 0)),
]


def _tc_slice0(buf, pos_i, seg_i, typ_i, cat_tab, gamma, beta):
    return pl.pallas_call(
        _tc_body0,
        grid=(BLKS_S,),
        in_specs=[pl.BlockSpec((BLK, D), lambda i: (i, 0)),
                  _ID_SPEC, _ID_SPEC, _ID_SPEC, *_TAB_SPECS],
        out_specs=pl.BlockSpec((BLK, D), lambda i: (i, 0)),
        out_shape=jax.ShapeDtypeStruct((N, D), jnp.float32),
        input_output_aliases={0: 0},
    )(buf, pos_i, seg_i, typ_i, cat_tab, gamma, beta)


def _tc_slice(s, buf, g_s, pos_i, seg_i, typ_i, cat_tab, gamma, beta):
    return pl.pallas_call(
        _tc_body_rest,
        grid=(BLKS_S,),
        in_specs=[pl.BlockSpec(memory_space=pl.ANY),
                  pl.BlockSpec((BLK, D), lambda i: (i, 0)),
                  _ID_SPEC, _ID_SPEC, _ID_SPEC, *_TAB_SPECS],
        out_specs=pl.BlockSpec((BLK, D), lambda i, s=s: (s * BLKS_S + i, 0)),
        out_shape=jax.ShapeDtypeStruct((N, D), jnp.float32),
        input_output_aliases={0: 0},
    )(buf, g_s, pos_i, seg_i, typ_i, cat_tab, gamma, beta)


def kernel(ids, src, seg, type, concept_ent_pairs, edge_idx, pos, need_gnn,
           word_table, token_type_table, pos_table, seg_table, gamma, beta):
    src_w = src.reshape(S, NW, NCHUNK, CHUNK).astype(jnp.int32)
    cat_tab = jnp.concatenate([pos_table, seg_table, token_type_table],
                              axis=0).astype(jnp.bfloat16)
    pos_i = pos.reshape(S, BLKS_S, 1, BLK).astype(jnp.int32)
    seg_i = seg.reshape(S, BLKS_S, 1, BLK).astype(jnp.int32)
    typ_i = type.reshape(S, BLKS_S, 1, BLK).astype(jnp.int32)
    gam = gamma.reshape(1, D)
    bet = beta.reshape(1, D)

    gs = [_sc_gather_word(src_w[s], word_table, N if s == 0 else NSL)
          for s in range(S)]
    buf = _tc_slice0(gs[0], pos_i[0], seg_i[0], typ_i[0], cat_tab, gam, bet)
    for s in range(1, S):
        buf = _tc_slice(s, buf, gs[s], pos_i[s], seg_i[s], typ_i[s],
                        cat_tab, gam, bet)
    return buf.reshape(B, L, D)


# speedup vs baseline: 1.0756x; 1.0059x over previous
import functools

import jax
import jax.numpy as jnp
from jax import lax
from jax.experimental import pallas as pl
from jax.experimental.pallas import tpu as pltpu
from jax.experimental.pallas import tpu_sc as plsc

B, L, D, V = 64, 512, 768, 100000
N = B * L
NC, NS = 2, 16
NW = NC * NS
CHUNK = 64

S = 2
NSL = N // S
TOK_W = NSL // NW
NCHUNK = TOK_W // CHUNK

BLK = 2048
BLKS_S = NSL // BLK
K_CAT = 512 + 3 + 21


def _sc_gather_word(src_w, word_table, out_rows):
    mesh = plsc.VectorSubcoreMesh(core_axis_name="c", subcore_axis_name="s")

    @functools.partial(
        pl.kernel,
        out_type=jax.ShapeDtypeStruct((out_rows, D), jnp.float32),
        mesh=mesh,
        scratch_types=[
            pltpu.VMEM((NCHUNK, CHUNK), jnp.int32),
            pltpu.VMEM((CHUNK, D), jnp.float32),
            pltpu.VMEM((CHUNK, D), jnp.float32),
            pltpu.SemaphoreType.DMA,
            pltpu.SemaphoreType.DMA,
        ],
    )
    def gather_kernel(src_hbm, tab_hbm, out_hbm, idx_v, buf0, buf1, sem0, sem1):
        wid = lax.axis_index("s") * NC + lax.axis_index("c")
        base = wid * TOK_W
        pltpu.sync_copy(src_hbm.at[wid], idx_v)
        bufs = (buf0, buf1)
        sems = (sem0, sem1)

        pltpu.async_copy(tab_hbm.at[idx_v.at[0]], buf0, sem0)

        def body(j, _):
            slot = lax.rem(j, 2)
            nslot = lax.rem(j + 1, 2)

            @pl.when(j + 1 < NCHUNK)
            def _():
                def fire(s):
                    pltpu.async_copy(tab_hbm.at[idx_v.at[j + 1]], bufs[s], sems[s])
                lax.cond(nslot == 0, lambda: fire(0), lambda: fire(1))

            def drain(s):
                pltpu.make_async_copy(tab_hbm.at[idx_v.at[j]], bufs[s], sems[s]).wait()
                pltpu.sync_copy(bufs[s], out_hbm.at[pl.ds(base + j * CHUNK, CHUNK)])
            lax.cond(slot == 0, lambda: drain(0), lambda: drain(1))
            return 0

        lax.fori_loop(0, NCHUNK, body, 0)

    return gather_kernel(src_w, word_table)


def _emb_ln(g, posb, segb, typb, tab, gam, bet):
    k_iota = lax.broadcasted_iota(jnp.int32, (K_CAT, BLK), 0)
    oh = ((k_iota == posb) | (k_iota == segb + 512) | (k_iota == typb + 515))
    oh = oh.astype(jnp.bfloat16)
    small = lax.dot_general(oh, tab, (((0,), (0,)), ((), ())),
                            preferred_element_type=jnp.float32)
    x = g + small
    mean = jnp.mean(x, axis=1, keepdims=True)
    xc = x - mean
    var = jnp.mean(xc * xc, axis=1, keepdims=True)
    y = xc * lax.rsqrt(var + 1e-6)
    return y * gam + bet


def _tc_body0(g_r, pos_r, seg_r, typ_r, tab_r, gam_r, bet_r, out_r):
    out_r[...] = _emb_ln(g_r[...], pos_r[0], seg_r[0], typ_r[0],
                         tab_r[...], gam_r[...], bet_r[...])


def _tc_body_rest(buf_r, g_r, pos_r, seg_r, typ_r, tab_r, gam_r, bet_r, out_r):
    del buf_r
    out_r[...] = _emb_ln(g_r[...], pos_r[0], seg_r[0], typ_r[0],
                         tab_r[...], gam_r[...], bet_r[...])


_ID_SPEC = pl.BlockSpec((1, 1, BLK), lambda i: (i, 0, 0))
_TAB_SPECS = [
    pl.BlockSpec((K_CAT, D), lambda i: (0, 0)),
    pl.BlockSpec((1, D), lambda i: (0, 0)),
    pl.BlockSpec((1, D), lambda i: (0, 0)),
]


def _tc_slice0(buf, pos_i, seg_i, typ_i, cat_tab, gamma, beta):
    return pl.pallas_call(
        _tc_body0,
        grid=(BLKS_S,),
        in_specs=[pl.BlockSpec((BLK, D), lambda i: (i, 0)),
                  _ID_SPEC, _ID_SPEC, _ID_SPEC, *_TAB_SPECS],
        out_specs=pl.BlockSpec((BLK, D), lambda i: (i, 0)),
        out_shape=jax.ShapeDtypeStruct((N, D), jnp.float32),
        input_output_aliases={0: 0},
    )(buf, pos_i, seg_i, typ_i, cat_tab, gamma, beta)


def _tc_slice(s, buf, g_s, pos_i, seg_i, typ_i, cat_tab, gamma, beta):
    return pl.pallas_call(
        _tc_body_rest,
        grid=(BLKS_S,),
        in_specs=[pl.BlockSpec(memory_space=pl.ANY),
                  pl.BlockSpec((BLK, D), lambda i: (i, 0)),
                  _ID_SPEC, _ID_SPEC, _ID_SPEC, *_TAB_SPECS],
        out_specs=pl.BlockSpec((BLK, D), lambda i, s=s: (s * BLKS_S + i, 0)),
        out_shape=jax.ShapeDtypeStruct((N, D), jnp.float32),
        input_output_aliases={0: 0},
    )(buf, g_s, pos_i, seg_i, typ_i, cat_tab, gamma, beta)


def kernel(ids, src, seg, type, concept_ent_pairs, edge_idx, pos, need_gnn,
           word_table, token_type_table, pos_table, seg_table, gamma, beta):
    src_w = src.reshape(S, NW, NCHUNK, CHUNK).astype(jnp.int32)
    cat_tab = jnp.concatenate([pos_table, seg_table, token_type_table],
                              axis=0).astype(jnp.bfloat16)
    pos_i = pos.reshape(S, BLKS_S, 1, BLK).astype(jnp.int32)
    seg_i = seg.reshape(S, BLKS_S, 1, BLK).astype(jnp.int32)
    typ_i = type.reshape(S, BLKS_S, 1, BLK).astype(jnp.int32)
    gam = gamma.reshape(1, D)
    bet = beta.reshape(1, D)

    gs = [_sc_gather_word(src_w[s], word_table, N if s == 0 else NSL)
          for s in range(S)]
    buf = _tc_slice0(gs[0], pos_i[0], seg_i[0], typ_i[0], cat_tab, gam, bet)
    for s in range(1, S):
        buf = _tc_slice(s, buf, gs[s], pos_i[s], seg_i[s], typ_i[s],
                        cat_tab, gam, bet)
    return buf.reshape(B, L, D)


# speedup vs baseline: 1.0897x; 1.0131x over previous
import functools

import jax
import jax.numpy as jnp
from jax import lax
from jax.experimental import pallas as pl
from jax.experimental.pallas import tpu as pltpu
from jax.experimental.pallas import tpu_sc as plsc

B, L, D, V = 64, 512, 768, 100000
N = B * L
NC, NS = 2, 16
NW = NC * NS
CHUNK = 64

S = 2
NSL = N // S
TOK_W = NSL // NW
NCHUNK = TOK_W // CHUNK

BLK = 2048
BLKS_S = NSL // BLK
K_CAT = 512 + 3 + 21


HALF = D // 2


def _sc_gather_word(src_w, word_table, out_rows):
    mesh = plsc.VectorSubcoreMesh(core_axis_name="c", subcore_axis_name="s")

    @functools.partial(
        pl.kernel,
        out_type=jax.ShapeDtypeStruct((out_rows, D), jnp.float32),
        mesh=mesh,
        scratch_types=[
            pltpu.VMEM((NCHUNK, CHUNK), jnp.int32),
            pltpu.VMEM((CHUNK, D), jnp.float32),
            pltpu.VMEM((CHUNK, D), jnp.float32),
            pltpu.SemaphoreType.DMA,
            pltpu.SemaphoreType.DMA,
        ],
    )
    def gather_kernel(src_hbm, tab_hbm, out_hbm, idx_v, buf0, buf1,
                      sem0, sem1):
        wid = lax.axis_index("s") * NC + lax.axis_index("c")
        base = wid * TOK_W
        pltpu.sync_copy(src_hbm.at[wid], idx_v)
        bufs = (buf0, buf1)
        sems = (sem0, sem1)

        pltpu.async_copy(tab_hbm.at[idx_v.at[0]], buf0, sem0)

        def body(j, _):
            slot = lax.rem(j, 2)
            nslot = lax.rem(j + 1, 2)

            @pl.when(j + 1 < NCHUNK)
            def _():
                def fire(s):
                    pltpu.async_copy(tab_hbm.at[idx_v.at[j + 1]], bufs[s], sems[s])
                lax.cond(nslot == 0, lambda: fire(0), lambda: fire(1))

            def drain(s):
                pltpu.make_async_copy(tab_hbm.at[idx_v.at[j]], bufs[s], sems[s]).wait()
                pltpu.sync_copy(bufs[s], out_hbm.at[pl.ds(base + j * CHUNK, CHUNK)])
            lax.cond(slot == 0, lambda: drain(0), lambda: drain(1))
            return 0

        lax.fori_loop(0, NCHUNK, body, 0)

    return gather_kernel(src_w, word_table)


def _emb_ln(g, posb, segb, typb, ptab, sttab, gam, bet):
    p_iota = lax.broadcasted_iota(jnp.int32, (512, BLK), 0)
    oh_p = (p_iota == posb).astype(jnp.bfloat16)
    st_iota = lax.broadcasted_iota(jnp.int32, (32, BLK), 0)
    oh_st = ((st_iota == segb) | (st_iota == typb + 3)).astype(jnp.bfloat16)
    dn = (((0,), (0,)), ((), ()))
    small = (lax.dot_general(oh_p, ptab, dn, preferred_element_type=jnp.float32)
             + lax.dot_general(oh_st, sttab, dn,
                               preferred_element_type=jnp.float32))
    x = g + small
    mean = jnp.mean(x, axis=1, keepdims=True)
    ex2 = jnp.mean(x * x, axis=1, keepdims=True)
    r = lax.rsqrt(ex2 - mean * mean + 1e-6)
    a = r * gam
    return x * a + (bet - mean * a)


def _tc_body0(g_r, pos_r, seg_r, typ_r, ptab_r, sttab_r, gam_r, bet_r, out_r):
    out_r[...] = _emb_ln(g_r[...], pos_r[0], seg_r[0], typ_r[0],
                         ptab_r[...], sttab_r[...], gam_r[...], bet_r[...])


def _tc_body_rest(buf_r, g_r, pos_r, seg_r, typ_r, ptab_r, sttab_r, gam_r,
                  bet_r, out_r):
    del buf_r
    out_r[...] = _emb_ln(g_r[...], pos_r[0], seg_r[0], typ_r[0],
                         ptab_r[...], sttab_r[...], gam_r[...], bet_r[...])


_ID_SPEC = pl.BlockSpec((1, 1, BLK), lambda i: (i, 0, 0))
_TAB_SPECS = [
    pl.BlockSpec((512, D), lambda i: (0, 0)),
    pl.BlockSpec((32, D), lambda i: (0, 0)),
    pl.BlockSpec((1, D), lambda i: (0, 0)),
    pl.BlockSpec((1, D), lambda i: (0, 0)),
]


def _tc_slice0(g0, pos_i, seg_i, typ_i, ptab, sttab, gamma, beta):
    return pl.pallas_call(
        _tc_body0,
        grid=(BLKS_S,),
        in_specs=[pl.BlockSpec((BLK, D), lambda i: (i, 0)),
                  _ID_SPEC, _ID_SPEC, _ID_SPEC, *_TAB_SPECS],
        out_specs=pl.BlockSpec((BLK, D), lambda i: (i, 0)),
        out_shape=jax.ShapeDtypeStruct((N, D), jnp.float32),
    )(g0, pos_i, seg_i, typ_i, ptab, sttab, gamma, beta)


def _tc_slice(s, buf, g_s, pos_i, seg_i, typ_i, ptab, sttab, gamma, beta):
    return pl.pallas_call(
        _tc_body_rest,
        grid=(BLKS_S,),
        in_specs=[pl.BlockSpec(memory_space=pl.ANY),
                  pl.BlockSpec((BLK, D), lambda i: (i, 0)),
                  _ID_SPEC, _ID_SPEC, _ID_SPEC, *_TAB_SPECS],
        out_specs=pl.BlockSpec((BLK, D), lambda i, s=s: (s * BLKS_S + i, 0)),
        out_shape=jax.ShapeDtypeStruct((N, D), jnp.float32),
        input_output_aliases={0: 0},
    )(buf, g_s, pos_i, seg_i, typ_i, ptab, sttab, gamma, beta)


def kernel(ids, src, seg, type, concept_ent_pairs, edge_idx, pos, need_gnn,
           word_table, token_type_table, pos_table, seg_table, gamma, beta):
    src_w = src.reshape(S, NW, NCHUNK, CHUNK).astype(jnp.int32)
    ptab = pos_table.astype(jnp.bfloat16)
    sttab = jnp.concatenate(
        [seg_table, token_type_table, jnp.zeros((8, D), jnp.float32)],
        axis=0).astype(jnp.bfloat16)
    pos_i = pos.reshape(S, BLKS_S, 1, BLK).astype(jnp.int32)
    seg_i = seg.reshape(S, BLKS_S, 1, BLK).astype(jnp.int32)
    typ_i = type.reshape(S, BLKS_S, 1, BLK).astype(jnp.int32)
    gam = gamma.reshape(1, D)
    bet = beta.reshape(1, D)

    gs = [_sc_gather_word(src_w[s], word_table, NSL) for s in range(S)]
    buf = _tc_slice0(gs[0], pos_i[0], seg_i[0], typ_i[0], ptab, sttab,
                     gam, bet)
    for s in range(1, S):
        buf = _tc_slice(s, buf, gs[s], pos_i[s], seg_i[s], typ_i[s],
                        ptab, sttab, gam, bet)
    return buf.reshape(B, L, D)
